# Initial kernel scaffold; baseline (speedup 1.0000x reference)
import jax
import jax.numpy as jnp
from jax.experimental import pallas as pl


def kernel(inp, hr_coord, proj_coord, head_w, head_b, rb_w1, rb_b1, rb_w2, rb_b2, tail_w, tail_b, Wq, bq, Wk, bk, Wv, bv, Wo, bo, m_w0, m_b0, m_w1, m_b1, m_w2, m_b2, m_w3, m_b3, m_w4, m_b4):
    raise NotImplementedError("write your pallas kernel here")


# speedup vs baseline: 40.9881x; 40.9881x over previous
import functools

import jax
import jax.numpy as jnp
from jax.experimental import pallas as pl
from jax.experimental.pallas import tpu as pltpu

_N_RES = 2
_WIN_H = 7
_WIN_W = 7
_WIN_D = 2


def _mm(a, b):
    return jax.lax.dot_general(a, b, (((1,), (0,)), ((), ())))


def _encoder_body(D, H, W, C,
                  inp_ref, head_w_ref, head_b_ref,
                  rb_w1_ref, rb_b1_ref, rb_w2_ref, rb_b2_ref,
                  tail_w_ref, tail_b_ref, out_ref,
                  xp_scr, acc_scr, x_scr):
    N = D * H * W

    n_blk = 1
    d_blk = D // n_blk
    r_blk = d_blk * H * W

    def conv64(x2, w_at):
        xp_scr[:] = jnp.zeros((D + 2, H + 2, W + 2, C), dtype=jnp.float32)
        xp_scr[1:D + 1, 1:H + 1, 1:W + 1, :] = x2.reshape(D, H, W, C)
        acc_scr[:] = jnp.zeros((N, C), dtype=jnp.float32)

        def tap(t, _):
            k = t // n_blk
            b = t % n_blk
            kd = k // 9
            kh = (k // 3) % 3
            kw = k % 3
            s = xp_scr[pl.ds(kd + b * d_blk, d_blk),
                       pl.ds(kh, H), pl.ds(kw, W), :]
            acc_scr[pl.ds(b * r_blk, r_blk), :] += _mm(
                s.reshape(r_blk, C), w_at(k))
            return 0

        jax.lax.fori_loop(0, 27 * n_blk, tap, 0)

    conv64(inp_ref[:], lambda k: head_w_ref[k])
    x_scr[:] = acc_scr[:] + head_b_ref[:]
    out_ref[:] = x_scr[:]
    for i in range(_N_RES):
        conv64(out_ref[:], lambda k: rb_w1_ref[i, k])
        conv64(jnp.maximum(acc_scr[:] + rb_b1_ref[i], 0.0),
               lambda k: rb_w2_ref[i, k])
        out_ref[:] = out_ref[:] + acc_scr[:] + rb_b2_ref[i]
    conv64(out_ref[:], lambda k: tail_w_ref[k])
    out_ref[:] = x_scr[:] + acc_scr[:] + tail_b_ref[:]


def _sample_attn_mlp_body(
        D, H, W, C, Q,
        feat_ref,
        dq_ref, hq_ref, wq_ref, dp_ref, hp_ref, wp_ref,
        fdq_ref, fhq_ref, fwq_ref, fdp_ref, fhp_ref, fwp_ref,
        wq_w_ref, bq_ref, wk_w_ref, bk_ref, wv_w_ref, bv_ref,
        wo_w_ref, bo_ref,
        mw0_ref, mb0_ref, mw1_ref, mb1_ref, mw2_ref, mb2_ref,
        mw3_ref, mb3_ref, mw4_ref, mb4_ref,
        out_ref,
        qf_scr, s_scr, afeat_scr):
    bq1 = 16 if Q % 16 == 0 else 1

    def qf_body(i, _):
        for j in range(bq1):
            n = i * bq1 + j
            d0 = dq_ref[n]
            h0 = hq_ref[n]
            w0 = wq_ref[n]
            fd = fdq_ref[n]
            fh = fhq_ref[n]
            fw = fwq_ref[n]
            p = feat_ref[pl.ds(d0, 2), pl.ds(h0, 2), pl.ds(w0, 2), :]
            t = (1.0 - fd) * p[0] + fd * p[1]
            t = (1.0 - fh) * t[0] + fh * t[1]
            t = (1.0 - fw) * t[0] + fw * t[1]
            qf_scr[pl.ds(n, 1), :] = t.reshape(1, C)
        return 0

    jax.lax.fori_loop(0, Q // bq1, qf_body, 0)

    qf = qf_scr[:]
    q = _mm(qf, wq_w_ref[:]) + bq_ref[:]
    inv_sqrt_c = 1.0 / jnp.sqrt(jnp.float32(C))
    s_scr[:] = _mm(q, wk_w_ref[:].T) * inv_sqrt_c

    jd = jax.lax.broadcasted_iota(jnp.int32, (_WIN_D, _WIN_H, _WIN_W), 0)
    jh = jax.lax.broadcasted_iota(jnp.int32, (_WIN_D, _WIN_H, _WIN_W), 1)
    jw = jax.lax.broadcasted_iota(jnp.int32, (_WIN_D, _WIN_H, _WIN_W), 2)

    bq2 = 32 if Q % 32 == 0 else 1

    def attn_body(i, _):
        for j in range(bq2):
            n = i * bq2 + j
            fld = dp_ref[n]
            flh = hp_ref[n]
            flw = wp_ref[n]
            sd = jnp.maximum(fld, 0)
            sh = jnp.maximum(flh - 3, 0)
            sw = jnp.maximum(flw - 3, 0)
            kd_shift = sd - fld
            kh_shift = sh - (flh - 3)
            kw_shift = sw - (flw - 3)
            fd = fdp_ref[n]
            fh = fhp_ref[n]
            fw = fwp_ref[n]
            p = feat_ref[pl.ds(sd, 3), pl.ds(sh, 8), pl.ds(sw, 8), :]
            kd = (1.0 - fd) * p[0:2] + fd * p[1:3]
            kh = (1.0 - fh) * kd[:, 0:_WIN_H] + fh * kd[:, 1:_WIN_H + 1]
            kf = (1.0 - fw) * kh[:, :, 0:_WIN_W] + fw * kh[:, :, 1:_WIN_W + 1]
            md = jnp.where(jd == 0, kd_shift + 1,
                           jnp.where(jd <= (_WIN_D - 1) - kd_shift, 1, 0))
            mh = jnp.where(jh == 0, kh_shift + 1,
                           jnp.where(jh <= (_WIN_H - 1) - kh_shift, 1, 0))
            mw = jnp.where(jw == 0, kw_shift + 1,
                           jnp.where(jw <= (_WIN_W - 1) - kw_shift, 1, 0))
            mult = (md * mh * mw).astype(jnp.float32)
            srow = s_scr[pl.ds(n, 1), :].reshape(1, 1, 1, C)
            logits = jnp.sum(kf * srow, axis=-1)
            m = jnp.max(logits)
            e = jnp.exp(logits - m) * mult
            z = jnp.sum(e)
            af = jnp.sum(e[..., None] * kf, axis=(0, 1, 2)) / z
            afeat_scr[pl.ds(n, 1), :] = af.reshape(1, C)
        return 0

    jax.lax.fori_loop(0, Q // bq2, attn_body, 0)

    av = _mm(afeat_scr[:], wv_w_ref[:]) + bv_ref[:]
    x = qf + _mm(av, wo_w_ref[:]) + bo_ref[:]
    x = jnp.maximum(_mm(x, mw0_ref[:]) + mb0_ref[:], 0.0)
    x = jnp.maximum(_mm(x, mw1_ref[:]) + mb1_ref[:], 0.0)
    x = jnp.maximum(_mm(x, mw2_ref[:]) + mb2_ref[:], 0.0)
    x = jnp.maximum(_mm(x, mw3_ref[:]) + mb3_ref[:], 0.0)
    out_ref[:] = _mm(x, mw4_ref[:]) + mb4_ref[:]


def kernel(inp, hr_coord, proj_coord, head_w, head_b, rb_w1, rb_b1, rb_w2,
           rb_b2, tail_w, tail_b, Wq, bq, Wk, bk, Wv, bv, Wo, bo,
           m_w0, m_b0, m_w1, m_b1, m_w2, m_b2, m_w3, m_b3, m_w4, m_b4):
    B, _, D, H, W = inp.shape
    Q = hr_coord.shape[1]
    C = head_w.shape[0]

    def tw(w):
        ci, co = w.shape[1], w.shape[0]
        return jnp.transpose(w, (2, 3, 4, 1, 0)).reshape(27, ci, co)

    N = D * H * W
    x0 = jnp.concatenate(
        [inp[0, 0].reshape(N, 1), jnp.zeros((N, C - 1), jnp.float32)], axis=1)
    head27 = jnp.concatenate(
        [tw(head_w), jnp.zeros((27, C - 1, C), jnp.float32)], axis=1)

    enc = pl.pallas_call(
        functools.partial(_encoder_body, D, H, W, C),
        out_shape=jax.ShapeDtypeStruct((N, C), jnp.float32),
        scratch_shapes=[
            pltpu.VMEM((D + 2, H + 2, W + 2, C), jnp.float32),
            pltpu.VMEM((N, C), jnp.float32),
            pltpu.VMEM((N, C), jnp.float32),
        ],
    )
    feat = enc(x0, head27, head_b.reshape(1, C),
               jnp.stack([tw(rb_w1[i]) for i in range(_N_RES)]),
               rb_b1.reshape(_N_RES, 1, C),
               jnp.stack([tw(rb_w2[i]) for i in range(_N_RES)]),
               rb_b2.reshape(_N_RES, 1, C),
               tw(tail_w), tail_b.reshape(1, C)).reshape(D, H, W, C)

    feat_pad = jnp.pad(feat, ((0, 1), (0, 3), (0, 3), (0, 0)), mode='edge')
    Dp, Hp, Wp = D + 1, H + 3, W + 3

    def coords_q(c):
        d = (c[:, 0] + 1.0) * 0.5 * (D - 1)
        h = (c[:, 1] + 1.0) * 0.5 * (H - 1)
        w = (c[:, 2] + 1.0) * 0.5 * (W - 1)
        d0 = jnp.floor(d); h0 = jnp.floor(h); w0 = jnp.floor(w)
        fd = d - d0; fh = h - h0; fw = w - w0
        di = jnp.clip(d0.astype(jnp.int32), 0, Dp - 2)
        hi = jnp.clip(h0.astype(jnp.int32), 0, Hp - 2)
        wi = jnp.clip(w0.astype(jnp.int32), 0, Wp - 2)
        return di, hi, wi, fd, fh, fw

    def coords_p(c):
        d = (c[:, 0] + 1.0) * 0.5 * (D - 1) - 0.5
        h = (c[:, 1] + 1.0) * 0.5 * (H - 1)
        w = (c[:, 2] + 1.0) * 0.5 * (W - 1)
        d0 = jnp.floor(d); h0 = jnp.floor(h); w0 = jnp.floor(w)
        fd = d - d0; fh = h - h0; fw = w - w0
        di = jnp.clip(d0.astype(jnp.int32), -1, Dp - 3)
        hi = jnp.clip(h0.astype(jnp.int32), 0, Hp - 5)
        wi = jnp.clip(w0.astype(jnp.int32), 0, Wp - 5)
        return di, hi, wi, fd, fh, fw

    dq, hq, wq, fdq, fhq, fwq = coords_q(hr_coord[0])
    dp, hp, wp, fdp, fhp, fwp = coords_p(proj_coord[0])

    smem = pl.BlockSpec(memory_space=pltpu.SMEM)
    vmem = pl.BlockSpec(memory_space=pltpu.VMEM)

    samp = pl.pallas_call(
        functools.partial(_sample_attn_mlp_body, D, H, W, C, Q),
        in_specs=[vmem] + [smem] * 12 + [vmem] * 18,
        out_shape=jax.ShapeDtypeStruct((Q, 1), jnp.float32),
        scratch_shapes=[
            pltpu.VMEM((Q, C), jnp.float32),
            pltpu.VMEM((Q, C), jnp.float32),
            pltpu.VMEM((Q, C), jnp.float32),
        ],
    )
    hid = m_w0.shape[1]
    pred = samp(
        feat_pad,
        dq, hq, wq, dp, hp, wp, fdq, fhq, fwq, fdp, fhp, fwp,
        Wq, bq.reshape(1, C), Wk, bk.reshape(1, C), Wv, bv.reshape(1, C),
        Wo, bo.reshape(1, C),
        m_w0, m_b0.reshape(1, hid), m_w1, m_b1.reshape(1, hid),
        m_w2, m_b2.reshape(1, hid), m_w3, m_b3.reshape(1, hid),
        m_w4, m_b4.reshape(1, 1))

    pred = pred[None]
    mask = jnp.ones((B, Q), dtype=pred.dtype)
    return pred, mask
